# Initial kernel scaffold; baseline (speedup 1.0000x reference)
import jax
import jax.numpy as jnp
from jax.experimental import pallas as pl


def kernel(feat, offset, gamma, beta):
    raise NotImplementedError("write your pallas kernel here")


# speedup vs baseline: 6.4793x; 6.4793x over previous
import functools

import jax
import jax.numpy as jnp
from jax.experimental import pallas as pl
from jax.experimental.pallas import tpu as pltpu

N_TOK = 32768
N_SEG = 16
C = 512
EPS = 1e-06
BLK = 2048


def _onehot(offset_ref, step, blk, n_seg):
    row = step * blk + jax.lax.broadcasted_iota(jnp.int32, (blk, 1), 0)
    seg = jnp.zeros((blk, 1), jnp.int32)
    for b in range(n_seg):
        seg = seg + (row >= offset_ref[b]).astype(jnp.int32)
    cols = jax.lax.broadcasted_iota(jnp.int32, (blk, n_seg), 1)
    return (seg == cols).astype(jnp.float32)


def _pass1_kernel(offset_ref, feat_ref, rn_ref):
    i = pl.program_id(0)
    n = pl.num_programs(0)
    onehot = _onehot(offset_ref, i, BLK, N_SEG)
    f = feat_ref[...]
    part = jax.lax.dot_general(
        onehot, f * f, (((0,), (0,)), ((), ())),
        preferred_element_type=jnp.float32)

    @pl.when(i == 0)
    def _():
        rn_ref[...] = jnp.zeros_like(rn_ref)

    rn_ref[...] += part

    @pl.when(i == n - 1)
    def _():
        resp = jnp.sqrt(rn_ref[...])
        mean = jnp.mean(resp, axis=1, keepdims=True)
        rn_ref[...] = resp / (mean + EPS)


def _pass2_kernel(offset_ref, feat_ref, rn_ref, gamma_ref, beta_ref, out_ref):
    i = pl.program_id(0)
    onehot = _onehot(offset_ref, i, BLK, N_SEG)
    rn_tok = jax.lax.dot_general(
        onehot, rn_ref[...], (((1,), (0,)), ((), ())),
        preferred_element_type=jnp.float32)
    f = feat_ref[...]
    out_ref[...] = f * (1.0 + gamma_ref[...] * rn_tok) + beta_ref[...]


@jax.jit
def kernel(feat, offset, gamma, beta):
    nblk = N_TOK // BLK
    rn = pl.pallas_call(
        _pass1_kernel,
        grid=(nblk,),
        in_specs=[
            pl.BlockSpec(memory_space=pltpu.SMEM),
            pl.BlockSpec((BLK, C), lambda i: (i, 0)),
        ],
        out_specs=pl.BlockSpec((N_SEG, C), lambda i: (0, 0)),
        out_shape=jax.ShapeDtypeStruct((N_SEG, C), jnp.float32),
    )(offset, feat)
    out = pl.pallas_call(
        _pass2_kernel,
        grid=(nblk,),
        in_specs=[
            pl.BlockSpec(memory_space=pltpu.SMEM),
            pl.BlockSpec((BLK, C), lambda i: (i, 0)),
            pl.BlockSpec((N_SEG, C), lambda i: (0, 0)),
            pl.BlockSpec((1, C), lambda i: (0, 0)),
            pl.BlockSpec((1, C), lambda i: (0, 0)),
        ],
        out_specs=pl.BlockSpec((BLK, C), lambda i: (i, 0)),
        out_shape=jax.ShapeDtypeStruct((N_TOK, C), jnp.float32),
    )(offset, feat, rn, gamma, beta)
    return out
